# baseline (speedup 1.0000x reference)
import jax
import jax.numpy as jnp
from jax import lax
from jax.experimental import pallas as pl
from jax.experimental.pallas import tpu as pltpu
from jax.experimental.pallas import tpu_sc as plsc

_VOCAB = 100000
_DIM = 32
_BATCH = 16384
_C = 20
_N = 10
_ALPHA = 1.0
_BETA = 1.0

_NC = 2
_NS = 16
_NW = _NC * _NS
_S = 32
_K = _BATCH // (_NW * _S)
_L = 16
_SS = 48

_OFF_CTX = 0
_OFF_NEG = _S * _C
_OFF_CTR = _OFF_NEG + _S * _N
_OFF_MSK = _OFF_CTR + _S
_CW = 2048


def _sample_compute(s, ctx_rows, ctxcb_rows, neg_rows, cemb, ccb,
                    comb, scores_b):
    lo = pl.ds(0, _L)
    hi = pl.ds(_L, _L)
    iota = lax.broadcasted_iota(jnp.int32, (_L,), 0)
    lane15 = iota == (_L - 1)
    c0 = cemb[s, lo]
    c1 = cemb[s, hi]
    idxv = jnp.broadcast_to(s * _SS, (_L,)).astype(jnp.int32)
    fzero = jnp.zeros((_L,), jnp.float32)

    def put(off, p):
        cum = plsc.cumsum(p)
        plsc.store_scatter(scores_b, [idxv + off], cum, mask=lane15)

    mv0 = comb[pl.ds(_OFF_MSK + s * 32, _L)].astype(jnp.float32)
    mv1 = comb[pl.ds(_OFF_MSK + s * 32 + _L, _L)].astype(jnp.float32)

    dnums = lax.GatherDimensionNumbers(
        offset_dims=(), collapsed_slice_dims=(0,), start_index_map=(0,))

    def take16(v, cv):
        return lax.gather(v, cv[:, None], dnums, slice_sizes=(1,),
                          mode=lax.GatherScatterMode.PROMISE_IN_BOUNDS)

    def mval(c):
        cv = jnp.broadcast_to(c, (_L,)).astype(jnp.int32)
        a = take16(mv0, cv & (_L - 1))
        b = take16(mv1, cv & (_L - 1))
        return jnp.where(cv < _L, a, b)

    def pa(c4, carry):
        a0, a1 = carry
        for u in range(4):
            c = c4 * 4 + u
            r = s * _C + c
            m = mval(c)
            a0 = a0 + m * ctxcb_rows[r, lo]
            a1 = a1 + m * ctxcb_rows[r, hi]
        return a0, a1

    acc0, acc1 = lax.fori_loop(0, _C // 4, pa, (fzero, fzero))

    def pb(c4, carry):
        for u in range(4):
            c = c4 * 4 + u
            r = s * _C + c
            put(c, c0 * ctx_rows[r, lo] + c1 * ctx_rows[r, hi])
        return carry

    lax.fori_loop(0, _C // 4, pb, 0)

    put(40, acc0 * ccb[s, lo] + acc1 * ccb[s, hi])

    def pc(n2, carry):
        for u in range(2):
            n = n2 * 2 + u
            r = s * _N + n
            v0 = neg_rows[r, lo]
            v1 = neg_rows[r, hi]
            put(_C + n, c0 * v0 + c1 * v1)
            put(_C + _N + n, acc0 * v0 + acc1 * v1)
        return carry

    lax.fori_loop(0, _N // 2, pc, 0)


def _sc_body(ctr_tab, ctx_tab, comb_arr, o_scores,
             comb0, comb1, ctx_rows0, ctx_rows1, ctxcb_rows0, ctxcb_rows1,
             neg_rows0, neg_rows1, cemb0, cemb1, ccb0, ccb1,
             scores0, scores1, semi, semg0, semg1, semw):
    w = lax.axis_index("c") * _NS + lax.axis_index("s")
    combs = (comb0, comb1)
    ctxr = (ctx_rows0, ctx_rows1)
    ctxcbr = (ctxcb_rows0, ctxcb_rows1)
    negr = (neg_rows0, neg_rows1)
    cembs = (cemb0, cemb1)
    ccbs = (ccb0, ccb1)
    scores = (scores0, scores1)
    semg = (semg0, semg1)

    def g_copies(p):
        comb = combs[p]
        return [
            pltpu.async_copy(ctx_tab.at[comb.at[pl.ds(_OFF_CTX, _S * _C)]],
                             ctxr[p], semg[p]),
            pltpu.async_copy(ctr_tab.at[comb.at[pl.ds(_OFF_CTX, _S * _C)]],
                             ctxcbr[p], semg[p]),
            pltpu.async_copy(ctx_tab.at[comb.at[pl.ds(_OFF_NEG, _S * _N)]],
                             negr[p], semg[p]),
            pltpu.async_copy(ctr_tab.at[comb.at[pl.ds(_OFF_CTR, _S)]],
                             cembs[p], semg[p]),
            pltpu.async_copy(ctx_tab.at[comb.at[pl.ds(_OFF_CTR, _S)]],
                             ccbs[p], semg[p]),
        ]

    def compute(p):
        def sample_body(s, c2):
            _sample_compute(s, ctxr[p], ctxcbr[p], negr[p], cembs[p],
                            ccbs[p], combs[p], scores[p])
            return c2

        lax.fori_loop(0, _S, sample_body, 0)

    h_idx = {}
    h_g = {}
    h_wb = {}
    h_idx[0] = pltpu.async_copy(comb_arr.at[w, 0], combs[0], semi)
    h_idx[0].wait()
    h_g[0] = g_copies(0)
    h_idx[1] = pltpu.async_copy(comb_arr.at[w, 1], combs[1], semi)
    for k in range(_K):
        p = k % 2
        if k + 1 < _K:
            h_idx[k + 1].wait()
            h_g[k + 1] = g_copies((k + 1) % 2)
        for cp in h_g[k]:
            cp.wait()
        if k + 2 < _K:
            h_idx[k + 2] = pltpu.async_copy(comb_arr.at[w, k + 2],
                                            combs[p], semi)
        if k >= 2:
            h_wb[k - 2].wait()
        compute(p)
        h_wb[k] = pltpu.async_copy(scores[p], o_scores.at[w * _K + k], semw)
    h_wb[_K - 2].wait()
    h_wb[_K - 1].wait()


def _logsig(x):
    return jnp.minimum(x, 0.0) - jnp.log(1.0 + jnp.exp(-jnp.abs(x)))


_BLK = 1024


def _tc_loss_body(s_ref, mask_ref, sg_ref, cb_ref, tot_ref):
    i = pl.program_id(0)
    s = s_ref[...]
    mask = mask_ref[...]
    pos_sg = s[:, :_C]
    neg_sg = s[:, _C:_C + _N]
    cnt = jnp.clip(jnp.sum(mask, axis=1, keepdims=True), 1.0)
    neg_cb = s[:, _C + _N:_C + 2 * _N] / cnt
    pos_cb = s[:, 40] / cnt[:, 0]

    neg_loss_sg = jnp.sum(_logsig(-neg_sg), axis=1)
    sg_part = jnp.sum(mask * -(_logsig(pos_sg) + neg_loss_sg[:, None]))
    cb_part = -jnp.sum(_logsig(pos_cb) + jnp.sum(_logsig(-neg_cb), axis=1))

    sg_part = sg_part * (1.0 / (_BATCH * _C))
    cb_part = cb_part * (1.0 / _BATCH)
    zero = jnp.zeros((1, 1), jnp.float32)

    @pl.when(i == 0)
    def _():
        sg_ref[...] = zero
        cb_ref[...] = zero

    sg_ref[...] += sg_part.reshape(1, 1)
    cb_ref[...] += cb_part.reshape(1, 1)

    @pl.when(i == pl.num_programs(0) - 1)
    def _():
        tot_ref[...] = _ALPHA * sg_ref[...] + _BETA * cb_ref[...]


def kernel(center_table, context_table, center_ids, context_ids, context_mask, negative_ids):
    i32 = jnp.int32
    f32 = jnp.float32
    ctx_i = context_ids.astype(i32).reshape(_NW, _K, _S * _C)
    neg_i = negative_ids.astype(i32).reshape(_NW, _K, _S * _N)
    ctr_i = center_ids.astype(i32).reshape(_NW, _K, _S)
    mask_i = jnp.pad(context_mask.astype(i32),
                     ((0, 0), (0, 32 - _C))).reshape(_NW, _K, _S * 32)
    pad_i = jnp.zeros((_NW, _K, _CW - _OFF_MSK - _S * 32), i32)
    comb = jnp.concatenate([ctx_i, neg_i, ctr_i, mask_i, pad_i], axis=2)

    mesh = plsc.VectorSubcoreMesh(core_axis_name="c", subcore_axis_name="s")
    sc_scores = pl.kernel(
        _sc_body,
        mesh=mesh,
        compiler_params=pltpu.CompilerParams(use_tc_tiling_on_sc=False,
                                             needs_layout_passes=False),
        out_type=jax.ShapeDtypeStruct((_NW * _K, _S * _SS), f32),
        scratch_types=[
            pltpu.VMEM((_CW,), i32),
            pltpu.VMEM((_CW,), i32),
            pltpu.VMEM((_S * _C, _DIM), f32),
            pltpu.VMEM((_S * _C, _DIM), f32),
            pltpu.VMEM((_S * _C, _DIM), f32),
            pltpu.VMEM((_S * _C, _DIM), f32),
            pltpu.VMEM((_S * _N, _DIM), f32),
            pltpu.VMEM((_S * _N, _DIM), f32),
            pltpu.VMEM((_S, _DIM), f32),
            pltpu.VMEM((_S, _DIM), f32),
            pltpu.VMEM((_S, _DIM), f32),
            pltpu.VMEM((_S, _DIM), f32),
            pltpu.VMEM((_S * _SS,), f32),
            pltpu.VMEM((_S * _SS,), f32),
            pltpu.SemaphoreType.DMA,
            pltpu.SemaphoreType.DMA,
            pltpu.SemaphoreType.DMA,
            pltpu.SemaphoreType.DMA,
        ],
    )
    scores = sc_scores(center_table, context_table, comb)

    scores = scores.reshape(_BATCH, _SS)
    mask2 = context_mask.astype(f32)

    out1 = jax.ShapeDtypeStruct((1, 1), f32)
    sg, cb, tot = pl.pallas_call(
        _tc_loss_body,
        grid=(_BATCH // _BLK,),
        in_specs=[
            pl.BlockSpec((_BLK, _SS), lambda i: (i, 0)),
            pl.BlockSpec((_BLK, _C), lambda i: (i, 0)),
        ],
        out_specs=[
            pl.BlockSpec((1, 1), lambda i: (0, 0)),
            pl.BlockSpec((1, 1), lambda i: (0, 0)),
            pl.BlockSpec((1, 1), lambda i: (0, 0)),
        ],
        out_shape=[out1, out1, out1],
    )(scores, mask2)

    return (tot[0, 0], sg[0, 0], cb[0, 0])
